# Initial kernel scaffold; baseline (speedup 1.0000x reference)
import functools

import jax
import jax.numpy as jnp
from jax import lax
from jax.experimental import pallas as pl
from jax.experimental.pallas import tpu as pltpu
from jax.experimental.pallas import tpu_sc as plsc

N = 10000
E = 160000
D_IN = 256
D_H = 512

NC = 2
NS = 16
CHUNK = 80
EPT = E // NS
NCHUNK = EPT // CHUNK
NPT = N // NS
ZROWS = 125

_MESH = plsc.VectorSubcoreMesh(core_axis_name="c", subcore_axis_name="s")


def _deg_body(edge_hbm, ones_hbm, zeros_hbm, out_hbm, idx_v, ones_v,
              stage_v, acc):
    c = lax.axis_index("c")
    s = lax.axis_index("s")
    pltpu.sync_copy(ones_hbm, ones_v)
    pltpu.sync_copy(edge_hbm.at[pl.ds((c * NS + s) * NCHUNK, NCHUNK)], idx_v)
    pltpu.sync_copy(zeros_hbm, stage_v)

    def _zero(z, _):
        pltpu.sync_copy(stage_v, acc.at[pl.ds(s * NPT + z * ZROWS, ZROWS)])
        return 0

    lax.fori_loop(0, NPT // ZROWS, _zero, 0)
    plsc.subcore_barrier()

    def _accum(j, _):
        pltpu.sync_copy(ones_v, acc.at[idx_v.at[j]], add=True)
        return 0

    lax.fori_loop(0, NCHUNK, _accum, 0)
    plsc.subcore_barrier()

    def _out(z, _):
        base = s * NPT + z * ZROWS
        pltpu.sync_copy(acc.at[pl.ds(base, ZROWS)], stage_v)
        pltpu.sync_copy(stage_v, out_hbm.at[pl.ds(c * N + base, ZROWS)])
        return 0

    lax.fori_loop(0, NPT // ZROWS, _out, 0)


def _make_deg_kernel():
    return pl.kernel(
        _deg_body,
        out_type=jax.ShapeDtypeStruct((2 * N, 16), jnp.float32),
        mesh=_MESH,
        scratch_types=[
            pltpu.VMEM((NCHUNK, CHUNK), jnp.int32),
            pltpu.VMEM((CHUNK, 16), jnp.float32),
            pltpu.VMEM((ZROWS, 16), jnp.float32),
            pltpu.VMEM_SHARED((N, 16), jnp.float32),
        ],
    )


def _make_agg_kernel(num_blocks):
    passes = num_blocks // NC

    def body(table_hbm, src_hbm, dst_hbm, zeros_hbm, out_hbm,
             srcv, dstv, rows_v, stage_v, acc, sem):
        c = lax.axis_index("c")
        s = lax.axis_index("s")
        pltpu.sync_copy(dst_hbm.at[pl.ds(s * NCHUNK, NCHUNK)], dstv)
        for p in range(passes):
            blk = p * NC + c
            pltpu.sync_copy(zeros_hbm, stage_v)

            def _zero(z, _):
                pltpu.sync_copy(
                    stage_v, acc.at[pl.ds(s * NPT + z * ZROWS, ZROWS)])
                return 0

            lax.fori_loop(0, NPT // ZROWS, _zero, 0)
            pltpu.sync_copy(
                src_hbm.at[pl.ds((blk * NS + s) * NCHUNK, NCHUNK)], srcv)
            plsc.subcore_barrier()

            def _edge(j, _):
                pltpu.async_copy(table_hbm.at[srcv.at[j]], rows_v, sem).wait()
                pltpu.sync_copy(rows_v, acc.at[dstv.at[j]], add=True)
                return 0

            lax.fori_loop(0, NCHUNK, _edge, 0)
            plsc.subcore_barrier()

            def _out(z, _):
                base = s * NPT + z * ZROWS
                pltpu.sync_copy(acc.at[pl.ds(base, ZROWS)], stage_v)
                pltpu.sync_copy(
                    stage_v, out_hbm.at[pl.ds(blk * N + base, ZROWS)])
                return 0

            lax.fori_loop(0, NPT // ZROWS, _out, 0)

    return pl.kernel(
        body,
        out_type=jax.ShapeDtypeStruct((num_blocks * N, 128), jnp.float32),
        mesh=_MESH,
        scratch_types=[
            pltpu.VMEM((NCHUNK, CHUNK), jnp.int32),
            pltpu.VMEM((NCHUNK, CHUNK), jnp.int32),
            pltpu.VMEM((CHUNK, 128), jnp.float32),
            pltpu.VMEM((ZROWS, 128), jnp.float32),
            pltpu.VMEM_SHARED((N, 128), jnp.float32),
            pltpu.SemaphoreType.DMA,
        ],
    )


_BN = 1000


def _scale_body(x_ref, degs_ref, out_ref):
    ns = lax.rsqrt(jnp.maximum(degs_ref[:, :1], 1.0))
    h = x_ref[...] * ns
    out_ref[0] = h[:, :128]
    out_ref[1] = h[:, 128:]


def _scale0(node_feats, deg_src):
    return pl.pallas_call(
        _scale_body,
        grid=(N // _BN,),
        in_specs=[
            pl.BlockSpec((_BN, D_IN), lambda i: (i, 0)),
            pl.BlockSpec((_BN, 16), lambda i: (i, 0)),
        ],
        out_specs=pl.BlockSpec((2, _BN, 128), lambda i: (0, i, 0)),
        out_shape=jax.ShapeDtypeStruct((2, N, 128), jnp.float32),
    )(node_feats, deg_src)


def _mm_body(nblk, scale_out, agg_ref, w_ref, b_ref, degd_ref, degs_ref,
             out_ref):
    acc = jnp.dot(agg_ref[0], w_ref[0], preferred_element_type=jnp.float32)
    for k in range(1, nblk):
        acc += jnp.dot(agg_ref[k], w_ref[k],
                       preferred_element_type=jnp.float32)
    nd = lax.rsqrt(jnp.maximum(degd_ref[:, :1], 1.0))
    y = jnp.maximum(acc * nd + b_ref[0][None, :], 0.0)
    if scale_out:
        ns = lax.rsqrt(jnp.maximum(degs_ref[:, :1], 1.0))
        y = y * ns
        for k in range(4):
            out_ref[k] = y[:, k * 128:(k + 1) * 128]
    else:
        out_ref[...] = y


def _mm(agg, w, b, deg_dst, deg_src, nblk, scale_out):
    if scale_out:
        out_shape = jax.ShapeDtypeStruct((4, N, 128), jnp.float32)
        out_specs = pl.BlockSpec((4, _BN, 128), lambda i: (0, i, 0))
    else:
        out_shape = jax.ShapeDtypeStruct((N, D_H), jnp.float32)
        out_specs = pl.BlockSpec((_BN, D_H), lambda i: (i, 0))
    return pl.pallas_call(
        functools.partial(_mm_body, nblk, scale_out),
        grid=(N // _BN,),
        in_specs=[
            pl.BlockSpec((nblk, _BN, 128), lambda i: (0, i, 0)),
            pl.BlockSpec((nblk, 128, D_H), lambda i: (0, 0, 0)),
            pl.BlockSpec((1, D_H), lambda i: (0, 0)),
            pl.BlockSpec((_BN, 16), lambda i: (i, 0)),
            pl.BlockSpec((_BN, 16), lambda i: (i, 0)),
        ],
        out_specs=out_specs,
        out_shape=out_shape,
    )(agg, w, b, deg_dst, deg_src)


def kernel(node_feats, edge_index, W1, b1, W2, b2):
    src = edge_index[0]
    dst = edge_index[1]

    edge2d = edge_index.reshape(2 * NS * NCHUNK, CHUNK)
    dst2d = dst.reshape(NS * NCHUNK, CHUNK)
    off2 = (jnp.arange(2, dtype=jnp.int32) * N)[:, None]
    src_l1 = (src[None, :] + off2).reshape(2 * NS * NCHUNK, CHUNK)
    off4 = (jnp.arange(4, dtype=jnp.int32) * N)[:, None]
    src_l2 = (src[None, :] + off4).reshape(4 * NS * NCHUNK, CHUNK)

    ones16 = jnp.ones((CHUNK, 16), jnp.float32)
    zeros16 = jnp.zeros((ZROWS, 16), jnp.float32)
    zeros128 = jnp.zeros((ZROWS, 128), jnp.float32)

    degs = _make_deg_kernel()(edge2d, ones16, zeros16)
    deg_src = degs[:N]
    deg_dst = degs[N:]

    h1t = _scale0(node_feats, deg_src).reshape(2 * N, 128)
    agg1 = _make_agg_kernel(2)(h1t, src_l1, dst2d, zeros128).reshape(2, N, 128)

    w1r = W1.reshape(2, 128, D_H)
    h2t = _mm(agg1, w1r, b1.reshape(1, D_H), deg_dst, deg_src, 2, True)
    h2t = h2t.reshape(4 * N, 128)

    agg2 = _make_agg_kernel(4)(h2t, src_l2, dst2d, zeros128).reshape(4, N, 128)
    w2r = W2.reshape(4, 128, D_H)
    return _mm(agg2, w2r, b2.reshape(1, D_H), deg_dst, deg_src, 4, False)


# speedup vs baseline: 2.8188x; 2.8188x over previous
import functools

import jax
import jax.numpy as jnp
from jax import lax
from jax.experimental import pallas as pl
from jax.experimental.pallas import tpu as pltpu
from jax.experimental.pallas import tpu_sc as plsc

N = 10000
E = 160000
D_IN = 256
D_H = 512

NC = 2
NS = 16
CHUNK = 80
EPT = E // NS
NCHUNK = EPT // CHUNK
NPAD = 10240
NPT = NPAD // NS
CW = 64
B1 = D_IN // CW
B2 = D_H // CW

_MESH = plsc.VectorSubcoreMesh(core_axis_name="c", subcore_axis_name="s",
                               num_cores=NC, num_subcores=NS)


def _deg_body(edge_hbm, ones_hbm, zeros_hbm, out_hbm, idx_v, ones_v,
              stage_v, acc):
    c = lax.axis_index("c")
    s = lax.axis_index("s")
    pltpu.sync_copy(ones_hbm, ones_v)
    pltpu.sync_copy(edge_hbm.at[c * NS + s], idx_v)
    pltpu.sync_copy(zeros_hbm, stage_v)
    pltpu.sync_copy(stage_v, acc.at[pl.ds(s * NPT, NPT)])
    plsc.subcore_barrier()

    def _accum(j, _):
        pltpu.sync_copy(ones_v, acc.at[idx_v.at[j]], add=True)
        return 0

    lax.fori_loop(0, NCHUNK, _accum, 0)
    plsc.subcore_barrier()
    pltpu.sync_copy(acc.at[pl.ds(s * NPT, NPT)], stage_v)
    pltpu.sync_copy(stage_v, out_hbm.at[pl.ds(c * NPAD + s * NPT, NPT)])


def _make_deg_kernel():
    return pl.kernel(
        _deg_body,
        out_type=jax.ShapeDtypeStruct((2 * NPAD, 16), jnp.float32),
        mesh=_MESH,
        compiler_params=pltpu.CompilerParams(use_tc_tiling_on_sc=False),
        scratch_types=[
            pltpu.VMEM((NCHUNK, CHUNK), jnp.int32),
            pltpu.VMEM((CHUNK, 16), jnp.float32),
            pltpu.VMEM((NPT, 16), jnp.float32),
            pltpu.VMEM_SHARED((NPAD, 16), jnp.float32),
        ],
    )


def _make_agg_kernel(num_blocks):
    passes = num_blocks // NC

    def body(table_hbm, src_hbm, dst_hbm, zeros_hbm, out_hbm,
             srcv, dstv, rows_v, stage_v, acc, sem):
        c = lax.axis_index("c")
        s = lax.axis_index("s")
        pltpu.sync_copy(dst_hbm.at[s], dstv)
        for p in range(passes):
            blk = p * NC + c
            pltpu.sync_copy(zeros_hbm, stage_v)
            pltpu.sync_copy(stage_v, acc.at[pl.ds(s * NPT, NPT)])
            pltpu.sync_copy(src_hbm.at[blk * NS + s], srcv)
            plsc.subcore_barrier()

            def _edge(j, _):
                pltpu.async_copy(table_hbm.at[srcv.at[j]], rows_v, sem).wait()
                pltpu.sync_copy(rows_v, acc.at[dstv.at[j]], add=True)
                return 0

            lax.fori_loop(0, NCHUNK, _edge, 0)
            plsc.subcore_barrier()
            pltpu.sync_copy(acc.at[pl.ds(s * NPT, NPT)], stage_v)
            pltpu.sync_copy(
                stage_v, out_hbm.at[pl.ds(blk * NPAD + s * NPT, NPT)])

    return pl.kernel(
        body,
        out_type=jax.ShapeDtypeStruct((num_blocks * NPAD, CW), jnp.float32),
        mesh=_MESH,
        compiler_params=pltpu.CompilerParams(use_tc_tiling_on_sc=False),
        scratch_types=[
            pltpu.VMEM((NCHUNK, CHUNK), jnp.int32),
            pltpu.VMEM((NCHUNK, CHUNK), jnp.int32),
            pltpu.VMEM((CHUNK, CW), jnp.float32),
            pltpu.VMEM((NPT, CW), jnp.float32),
            pltpu.VMEM_SHARED((NPAD, CW), jnp.float32),
            pltpu.SemaphoreType.DMA,
        ],
    )


_BN = 400


def _scale_body(x_ref, degs_ref, out_ref):
    ns = lax.rsqrt(jnp.maximum(degs_ref[:, :1], 1.0))
    h = x_ref[...] * ns
    for k in range(B1):
        out_ref[k] = h[:, k * CW:(k + 1) * CW]


def _scale0(node_feats, deg_src):
    return pl.pallas_call(
        _scale_body,
        grid=(N // _BN,),
        in_specs=[
            pl.BlockSpec((_BN, D_IN), lambda i: (i, 0)),
            pl.BlockSpec((_BN, 16), lambda i: (i, 0)),
        ],
        out_specs=pl.BlockSpec((B1, _BN, CW), lambda i: (0, i, 0)),
        out_shape=jax.ShapeDtypeStruct((B1, NPAD, CW), jnp.float32),
    )(node_feats, deg_src)


def _mm_body(nblk, scale_out, agg_ref, w_ref, b_ref, degd_ref, degs_ref,
             out_ref):
    acc = jnp.dot(agg_ref[0], w_ref[0], preferred_element_type=jnp.float32)
    for k in range(1, nblk):
        acc += jnp.dot(agg_ref[k], w_ref[k],
                       preferred_element_type=jnp.float32)
    nd = lax.rsqrt(jnp.maximum(degd_ref[:, :1], 1.0))
    y = jnp.maximum(acc * nd + b_ref[0][None, :], 0.0)
    if scale_out:
        ns = lax.rsqrt(jnp.maximum(degs_ref[:, :1], 1.0))
        y = y * ns
        for k in range(B2):
            out_ref[k] = y[:, k * CW:(k + 1) * CW]
    else:
        out_ref[...] = y


def _mm(agg, w, b, deg_dst, deg_src, nblk, scale_out):
    if scale_out:
        out_shape = jax.ShapeDtypeStruct((B2, NPAD, CW), jnp.float32)
        out_specs = pl.BlockSpec((B2, _BN, CW), lambda i: (0, i, 0))
    else:
        out_shape = jax.ShapeDtypeStruct((N, D_H), jnp.float32)
        out_specs = pl.BlockSpec((_BN, D_H), lambda i: (i, 0))
    return pl.pallas_call(
        functools.partial(_mm_body, nblk, scale_out),
        grid=(N // _BN,),
        in_specs=[
            pl.BlockSpec((nblk, _BN, CW), lambda i: (0, i, 0)),
            pl.BlockSpec((nblk, CW, D_H), lambda i: (0, 0, 0)),
            pl.BlockSpec((1, D_H), lambda i: (0, 0)),
            pl.BlockSpec((_BN, 16), lambda i: (i, 0)),
            pl.BlockSpec((_BN, 16), lambda i: (i, 0)),
        ],
        out_specs=out_specs,
        out_shape=out_shape,
    )(agg, w, b, deg_dst, deg_src)


def kernel(node_feats, edge_index, W1, b1, W2, b2):
    src = edge_index[0]
    dst = edge_index[1]

    edge3d = edge_index.reshape(2 * NS, NCHUNK, CHUNK)
    dst3d = dst.reshape(NS, NCHUNK, CHUNK)
    off1 = (jnp.arange(B1, dtype=jnp.int32) * NPAD)[:, None]
    src_l1 = (src[None, :] + off1).reshape(B1 * NS, NCHUNK, CHUNK)
    off2 = (jnp.arange(B2, dtype=jnp.int32) * NPAD)[:, None]
    src_l2 = (src[None, :] + off2).reshape(B2 * NS, NCHUNK, CHUNK)

    ones16 = jnp.ones((CHUNK, 16), jnp.float32)
    zeros16 = jnp.zeros((NPT, 16), jnp.float32)
    zerosw = jnp.zeros((NPT, CW), jnp.float32)

    degs = _make_deg_kernel()(edge3d, ones16, zeros16)
    deg_src = degs[:N]
    deg_dst = degs[NPAD:NPAD + N]

    h1t = _scale0(node_feats, deg_src).reshape(B1 * NPAD, CW)
    agg1 = _make_agg_kernel(B1)(h1t, src_l1, dst3d, zerosw)
    agg1 = agg1.reshape(B1, NPAD, CW)

    w1r = W1.reshape(B1, CW, D_H)
    h2t = _mm(agg1, w1r, b1.reshape(1, D_H), deg_dst, deg_src, B1, True)
    h2t = h2t.reshape(B2 * NPAD, CW)

    agg2 = _make_agg_kernel(B2)(h2t, src_l2, dst3d, zerosw)
    agg2 = agg2.reshape(B2, NPAD, CW)
    w2r = W2.reshape(B2, CW, D_H)
    return _mm(agg2, w2r, b2.reshape(1, D_H), deg_dst, deg_src, B2, False)


# speedup vs baseline: 4.0222x; 1.4269x over previous
import functools

import jax
import jax.numpy as jnp
from jax import lax
from jax.experimental import pallas as pl
from jax.experimental.pallas import tpu as pltpu
from jax.experimental.pallas import tpu_sc as plsc

N = 10000
E = 160000
D_IN = 256
D_H = 512

NC = 2
NS = 16
CHUNK = 125
EPT = E // NS
NCHUNK = EPT // CHUNK
PIPE = 2
NPAD = 10240
NPT = NPAD // NS
CW = 64
B1 = D_IN // CW
B2 = D_H // CW

_MESH = plsc.VectorSubcoreMesh(core_axis_name="c", subcore_axis_name="s",
                               num_cores=NC, num_subcores=NS)


def _deg_body(edge_hbm, ones_hbm, zeros_hbm, out_hbm, idx_v, ones_v,
              stage_v, acc):
    c = lax.axis_index("c")
    s = lax.axis_index("s")
    pltpu.sync_copy(ones_hbm, ones_v)
    pltpu.sync_copy(edge_hbm.at[c * NS + s], idx_v)
    pltpu.sync_copy(zeros_hbm, stage_v)
    pltpu.sync_copy(stage_v, acc.at[pl.ds(s * NPT, NPT)])
    plsc.subcore_barrier()

    def _accum(j, _):
        pltpu.sync_copy(ones_v, acc.at[idx_v.at[j]], add=True)
        return 0

    lax.fori_loop(0, NCHUNK, _accum, 0)
    plsc.subcore_barrier()
    pltpu.sync_copy(acc.at[pl.ds(s * NPT, NPT)], stage_v)
    pltpu.sync_copy(stage_v, out_hbm.at[pl.ds(c * NPAD + s * NPT, NPT)])


def _make_deg_kernel():
    return pl.kernel(
        _deg_body,
        out_type=jax.ShapeDtypeStruct((2 * NPAD, 16), jnp.float32),
        mesh=_MESH,
        compiler_params=pltpu.CompilerParams(use_tc_tiling_on_sc=False),
        scratch_types=[
            pltpu.VMEM((NCHUNK, CHUNK), jnp.int32),
            pltpu.VMEM((CHUNK, 16), jnp.float32),
            pltpu.VMEM((NPT, 16), jnp.float32),
            pltpu.VMEM_SHARED((NPAD, 16), jnp.float32),
        ],
    )


def _make_agg_kernel(num_blocks):
    passes = num_blocks // NC

    def body(table_hbm, src_hbm, dst_hbm, zeros_hbm, out_hbm,
             srcv, dstv, rows_v, stage_v, acc, gsems, ssems):
        c = lax.axis_index("c")
        s = lax.axis_index("s")
        pltpu.sync_copy(dst_hbm.at[s], dstv)

        def _gather(j, b):
            return pltpu.async_copy(
                table_hbm.at[srcv.at[j]], rows_v.at[b], gsems.at[b])

        def _scatter(j, b):
            return pltpu.async_copy(
                rows_v.at[b], acc.at[dstv.at[j]], ssems.at[b], add=True)

        for p in range(passes):
            blk = p * NC + c
            pltpu.sync_copy(zeros_hbm, stage_v)
            pltpu.sync_copy(stage_v, acc.at[pl.ds(s * NPT, NPT)])
            pltpu.sync_copy(src_hbm.at[blk * NS + s], srcv)
            plsc.subcore_barrier()

            for b in range(PIPE - 1):
                _gather(b, b)

            def _group(g, _):
                for b in range(PIPE):
                    j = g * PIPE + b
                    pltpu.make_async_copy(
                        table_hbm.at[srcv.at[j]], rows_v.at[b],
                        gsems.at[b]).wait()
                    _scatter(j, b)
                    jn = j + PIPE - 1
                    pb = (b + PIPE - 1) % PIPE

                    @pl.when(jnp.logical_and(j > 0, jn < NCHUNK))
                    def _():
                        pltpu.make_async_copy(
                            rows_v.at[pb], acc.at[dstv.at[j - 1]],
                            ssems.at[pb]).wait()

                    @pl.when(jn < NCHUNK)
                    def _():
                        _gather(jn, pb)
                return 0

            lax.fori_loop(0, NCHUNK // PIPE, _group, 0)
            for b in range(PIPE):
                j = NCHUNK - PIPE + b
                pltpu.make_async_copy(
                    rows_v.at[b], acc.at[dstv.at[j]], ssems.at[b]).wait()
            plsc.subcore_barrier()
            pltpu.sync_copy(acc.at[pl.ds(s * NPT, NPT)], stage_v)
            pltpu.sync_copy(
                stage_v, out_hbm.at[pl.ds(blk * NPAD + s * NPT, NPT)])

    return pl.kernel(
        body,
        out_type=jax.ShapeDtypeStruct((num_blocks * NPAD, CW), jnp.float32),
        mesh=_MESH,
        compiler_params=pltpu.CompilerParams(use_tc_tiling_on_sc=False),
        scratch_types=[
            pltpu.VMEM((NCHUNK, CHUNK), jnp.int32),
            pltpu.VMEM((NCHUNK, CHUNK), jnp.int32),
            pltpu.VMEM((PIPE, CHUNK, CW), jnp.float32),
            pltpu.VMEM((NPT, CW), jnp.float32),
            pltpu.VMEM_SHARED((NPAD, CW), jnp.float32),
            pltpu.SemaphoreType.DMA((PIPE,)),
            pltpu.SemaphoreType.DMA((PIPE,)),
        ],
    )


_BN = 400


def _scale_body(x_ref, degs_ref, out_ref):
    ns = lax.rsqrt(jnp.maximum(degs_ref[:, :1], 1.0))
    h = x_ref[...] * ns
    for k in range(B1):
        out_ref[k] = h[:, k * CW:(k + 1) * CW]


def _scale0(node_feats, deg_src):
    return pl.pallas_call(
        _scale_body,
        grid=(N // _BN,),
        in_specs=[
            pl.BlockSpec((_BN, D_IN), lambda i: (i, 0)),
            pl.BlockSpec((_BN, 16), lambda i: (i, 0)),
        ],
        out_specs=pl.BlockSpec((B1, _BN, CW), lambda i: (0, i, 0)),
        out_shape=jax.ShapeDtypeStruct((B1, NPAD, CW), jnp.float32),
    )(node_feats, deg_src)


def _mm_body(nblk, scale_out, agg_ref, w_ref, b_ref, degd_ref, degs_ref,
             out_ref):
    acc = jnp.dot(agg_ref[0], w_ref[0], preferred_element_type=jnp.float32)
    for k in range(1, nblk):
        acc += jnp.dot(agg_ref[k], w_ref[k],
                       preferred_element_type=jnp.float32)
    nd = lax.rsqrt(jnp.maximum(degd_ref[:, :1], 1.0))
    y = jnp.maximum(acc * nd + b_ref[0][None, :], 0.0)
    if scale_out:
        ns = lax.rsqrt(jnp.maximum(degs_ref[:, :1], 1.0))
        y = y * ns
        for k in range(B2):
            out_ref[k] = y[:, k * CW:(k + 1) * CW]
    else:
        out_ref[...] = y


def _mm(agg, w, b, deg_dst, deg_src, nblk, scale_out):
    if scale_out:
        out_shape = jax.ShapeDtypeStruct((B2, NPAD, CW), jnp.float32)
        out_specs = pl.BlockSpec((B2, _BN, CW), lambda i: (0, i, 0))
    else:
        out_shape = jax.ShapeDtypeStruct((N, D_H), jnp.float32)
        out_specs = pl.BlockSpec((_BN, D_H), lambda i: (i, 0))
    return pl.pallas_call(
        functools.partial(_mm_body, nblk, scale_out),
        grid=(N // _BN,),
        in_specs=[
            pl.BlockSpec((nblk, _BN, CW), lambda i: (0, i, 0)),
            pl.BlockSpec((nblk, CW, D_H), lambda i: (0, 0, 0)),
            pl.BlockSpec((1, D_H), lambda i: (0, 0)),
            pl.BlockSpec((_BN, 16), lambda i: (i, 0)),
            pl.BlockSpec((_BN, 16), lambda i: (i, 0)),
        ],
        out_specs=out_specs,
        out_shape=out_shape,
    )(agg, w, b, deg_dst, deg_src)


def kernel(node_feats, edge_index, W1, b1, W2, b2):
    src = edge_index[0]
    dst = edge_index[1]

    edge3d = edge_index.reshape(2 * NS, NCHUNK, CHUNK)
    dst3d = dst.reshape(NS, NCHUNK, CHUNK)
    off1 = (jnp.arange(B1, dtype=jnp.int32) * NPAD)[:, None]
    src_l1 = (src[None, :] + off1).reshape(B1 * NS, NCHUNK, CHUNK)
    off2 = (jnp.arange(B2, dtype=jnp.int32) * NPAD)[:, None]
    src_l2 = (src[None, :] + off2).reshape(B2 * NS, NCHUNK, CHUNK)

    ones16 = jnp.ones((CHUNK, 16), jnp.float32)
    zeros16 = jnp.zeros((NPT, 16), jnp.float32)
    zerosw = jnp.zeros((NPT, CW), jnp.float32)

    degs = _make_deg_kernel()(edge3d, ones16, zeros16)
    deg_src = degs[:N]
    deg_dst = degs[NPAD:NPAD + N]

    h1t = _scale0(node_feats, deg_src).reshape(B1 * NPAD, CW)
    agg1 = _make_agg_kernel(B1)(h1t, src_l1, dst3d, zerosw)
    agg1 = agg1.reshape(B1, NPAD, CW)

    w1r = W1.reshape(B1, CW, D_H)
    h2t = _mm(agg1, w1r, b1.reshape(1, D_H), deg_dst, deg_src, B1, True)
    h2t = h2t.reshape(B2 * NPAD, CW)

    agg2 = _make_agg_kernel(B2)(h2t, src_l2, dst3d, zerosw)
    agg2 = agg2.reshape(B2, NPAD, CW)
    w2r = W2.reshape(B2, CW, D_H)
    return _mm(agg2, w2r, b2.reshape(1, D_H), deg_dst, deg_src, B2, False)
